# baseline (speedup 1.0000x reference)
import jax
import jax.numpy as jnp
from jax.experimental import pallas as pl
from jax.experimental.pallas import tpu as pltpu

_THRESH = 0.01
_S = 1024
_D = 256
_M = 2
_PB = 2


def _gconv_body(x_ref, adj_ref, w00_ref, w01_ref, w10_ref, w11_ref,
                b0_ref, b1_ref, wo_ref, bo_ref, out_ref,
                ah_ref, ws_ref, wob_ref, hcat_ref, ybuf_ref):
    j = pl.program_id(0)
    bf = jnp.bfloat16

    @pl.when(j == 0)
    def _build():
        rows = jax.lax.broadcasted_iota(jnp.int32, (_S, _S), 0)
        cols = jax.lax.broadcasted_iota(jnp.int32, (_S, _S), 1)
        eye = jnp.where(rows == cols, jnp.float32(1.0), jnp.float32(0.0))
        ones = jnp.ones((_S, 128), jnp.float32)
        for m in range(_M):
            a = adj_ref[m]
            sym = jnp.where(rows >= cols, a, a.T)
            sa = jnp.abs(sym)
            c = jnp.where(sa > _THRESH, sa, jnp.float32(0.0))
            deg = jnp.dot(c, ones, preferred_element_type=jnp.float32)[:, 0:1] + 1.0
            dinv = jnp.where(deg > 0.0, jax.lax.rsqrt(deg), jnp.float32(0.0))
            ah_ref[m] = (dinv * (c + eye) * dinv.reshape(1, _S)).astype(bf)
        ws_ref[0] = (w00_ref[:] + w01_ref[:]).astype(bf)
        ws_ref[1] = (w10_ref[:] + w11_ref[:]).astype(bf)
        wob_ref[:] = wo_ref[:].astype(bf)

    xc = x_ref[:].reshape(_PB * _S, _D).astype(bf)
    for m in range(_M):
        h = jnp.dot(xc, ws_ref[m], preferred_element_type=jnp.float32).astype(bf)
        for p in range(_PB):
            hcat_ref[m, :, p * _D:(p + 1) * _D] = h[p * _S:(p + 1) * _S]
        t = jnp.dot(ah_ref[m], hcat_ref[m], preferred_element_type=jnp.float32)
        bm = (b0_ref if m == 0 else b1_ref)[0][None, :]
        y = jnp.maximum(t + bm, 0.0).astype(bf)
        for p in range(_PB):
            ybuf_ref[p, :, m * _D:(m + 1) * _D] = y[:, p * _D:(p + 1) * _D]
    for p in range(_PB):
        o = jnp.dot(ybuf_ref[p], wob_ref[:], preferred_element_type=jnp.float32)
        o += bo_ref[0][None, :]
        out_ref[p] = jnp.maximum(o, 0.0)


def kernel(x, adj, W_0_0, b_0_0, W_0_1, b_0_1, W_1_0, b_1_0, W_1_1, b_1_1,
           W_out, b_out):
    B = x.shape[0]
    b0 = jnp.tile((b_0_0 + b_0_1).reshape(1, _D), (1, _PB))
    b1 = jnp.tile((b_1_0 + b_1_1).reshape(1, _D), (1, _PB))
    bo = b_out.reshape(1, _D)
    const3 = lambda *_: (0, 0, 0)
    const2 = lambda *_: (0, 0)
    return pl.pallas_call(
        _gconv_body,
        grid=(B // _PB,),
        in_specs=[
            pl.BlockSpec((_PB, _S, _D), lambda j: (j, 0, 0)),
            pl.BlockSpec((_M, _S, _S), const3),
            pl.BlockSpec((_D, _D), const2),
            pl.BlockSpec((_D, _D), const2),
            pl.BlockSpec((_D, _D), const2),
            pl.BlockSpec((_D, _D), const2),
            pl.BlockSpec((1, _PB * _D), const2),
            pl.BlockSpec((1, _PB * _D), const2),
            pl.BlockSpec((_M * _D, _D), const2),
            pl.BlockSpec((1, _D), const2),
        ],
        out_specs=pl.BlockSpec((_PB, _S, _D), lambda j: (j, 0, 0)),
        out_shape=jax.ShapeDtypeStruct((B, _S, _D), jnp.float32),
        scratch_shapes=[
            pltpu.VMEM((_M, _S, _S), jnp.bfloat16),
            pltpu.VMEM((_M, _D, _D), jnp.bfloat16),
            pltpu.VMEM((_M * _D, _D), jnp.bfloat16),
            pltpu.VMEM((_M, _S, _PB * _D), jnp.bfloat16),
            pltpu.VMEM((_PB, _S, _M * _D), jnp.bfloat16),
        ],
    )(x, adj, W_0_0, W_0_1, W_1_0, W_1_1, b0, b1, W_out, bo)


# speedup vs baseline: 1.0198x; 1.0198x over previous
import jax
import jax.numpy as jnp
from jax.experimental import pallas as pl
from jax.experimental.pallas import tpu as pltpu

_THRESH = 0.01
_S = 1024
_D = 256
_M = 2
_PB = 2


def _gconv_body(x_ref, adj_ref, w00_ref, w01_ref, w10_ref, w11_ref,
                b0_ref, b1_ref, wo_ref, bo_ref, out_ref,
                ah_ref, ws_ref, wob_ref):
    j = pl.program_id(0)
    bf = jnp.bfloat16

    @pl.when(j == 0)
    def _build():
        rows = jax.lax.broadcasted_iota(jnp.int32, (_S, _S), 0)
        cols = jax.lax.broadcasted_iota(jnp.int32, (_S, _S), 1)
        eye = jnp.where(rows == cols, jnp.float32(1.0), jnp.float32(0.0))
        for m in range(_M):
            a = adj_ref[m]
            sym = jnp.where(rows >= cols, a, a.T)
            sa = jnp.abs(sym)
            c = jnp.where(sa > _THRESH, sa, jnp.float32(0.0))
            deg = jnp.sum(c, axis=1) + 1.0
            dinv = jnp.where(deg > 0.0, jax.lax.rsqrt(deg), jnp.float32(0.0))
            ah_ref[m] = (dinv[:, None] * (c + eye) * dinv[None, :]).astype(bf)
        ws_ref[0] = (w00_ref[:] + w01_ref[:]).astype(bf)
        ws_ref[1] = (w10_ref[:] + w11_ref[:]).astype(bf)
        wob_ref[:] = wo_ref[:].astype(bf)

    xc = x_ref[:].reshape(_PB * _S, _D).astype(bf)
    ys = []
    for m in range(_M):
        h = jnp.dot(xc, ws_ref[m], preferred_element_type=jnp.float32).astype(bf)
        hcat = jnp.concatenate([h[i * _S:(i + 1) * _S] for i in range(_PB)], axis=1)
        t = jnp.dot(ah_ref[m], hcat, preferred_element_type=jnp.float32)
        bm = (b0_ref if m == 0 else b1_ref)[0][None, :]
        y = jnp.maximum(t + bm, 0.0).astype(bf)
        ys.append(y)
    for p in range(_PB):
        sl = slice(p * _D, (p + 1) * _D)
        o = jnp.dot(ys[0][:, sl], wob_ref[0:_D], preferred_element_type=jnp.float32)
        o += jnp.dot(ys[1][:, sl], wob_ref[_D:2 * _D], preferred_element_type=jnp.float32)
        o += bo_ref[0][None, :]
        out_ref[p] = jnp.maximum(o, 0.0)


def kernel(x, adj, W_0_0, b_0_0, W_0_1, b_0_1, W_1_0, b_1_0, W_1_1, b_1_1,
           W_out, b_out):
    B = x.shape[0]
    b0 = jnp.tile((b_0_0 + b_0_1).reshape(1, _D), (1, _PB))
    b1 = jnp.tile((b_1_0 + b_1_1).reshape(1, _D), (1, _PB))
    bo = b_out.reshape(1, _D)
    const3 = lambda *_: (0, 0, 0)
    const2 = lambda *_: (0, 0)
    return pl.pallas_call(
        _gconv_body,
        grid=(B // _PB,),
        in_specs=[
            pl.BlockSpec((_PB, _S, _D), lambda j: (j, 0, 0)),
            pl.BlockSpec((_M, _S, _S), const3),
            pl.BlockSpec((_D, _D), const2),
            pl.BlockSpec((_D, _D), const2),
            pl.BlockSpec((_D, _D), const2),
            pl.BlockSpec((_D, _D), const2),
            pl.BlockSpec((1, _PB * _D), const2),
            pl.BlockSpec((1, _PB * _D), const2),
            pl.BlockSpec((_M * _D, _D), const2),
            pl.BlockSpec((1, _D), const2),
        ],
        out_specs=pl.BlockSpec((_PB, _S, _D), lambda j: (j, 0, 0)),
        out_shape=jax.ShapeDtypeStruct((B, _S, _D), jnp.float32),
        scratch_shapes=[
            pltpu.VMEM((_M, _S, _S), jnp.bfloat16),
            pltpu.VMEM((_M, _D, _D), jnp.bfloat16),
            pltpu.VMEM((_M * _D, _D), jnp.bfloat16),
        ],
    )(x, adj, W_0_0, W_0_1, W_1_0, W_1_1, b0, b1, W_out, bo)


# speedup vs baseline: 1.0275x; 1.0076x over previous
import jax
import jax.numpy as jnp
from jax.experimental import pallas as pl
from jax.experimental.pallas import tpu as pltpu

_THRESH = 0.01
_S = 1024
_D = 256
_M = 2
_PB = 2


def _gconv_body(x_ref, adj_ref, w00_ref, w01_ref, w10_ref, w11_ref,
                b0_ref, b1_ref, wo_ref, bo_ref, out_ref,
                ah_ref, ws_ref, wob_ref, adjv_ref, sem0, sem1):
    j = pl.program_id(0)
    bf = jnp.bfloat16

    @pl.when(j == 0)
    def _build():
        cp0 = pltpu.make_async_copy(adj_ref.at[0], adjv_ref.at[0], sem0)
        cp1 = pltpu.make_async_copy(adj_ref.at[1], adjv_ref.at[1], sem1)
        cp0.start()
        cp1.start()
        ws_ref[0] = (w00_ref[:] + w01_ref[:]).astype(bf)
        ws_ref[1] = (w10_ref[:] + w11_ref[:]).astype(bf)
        wob_ref[:] = wo_ref[:].astype(bf)
        rows = jax.lax.broadcasted_iota(jnp.int32, (_S, _S), 0)
        cols = jax.lax.broadcasted_iota(jnp.int32, (_S, _S), 1)
        eye = jnp.where(rows == cols, jnp.float32(1.0), jnp.float32(0.0))
        for m in range(_M):
            (cp0 if m == 0 else cp1).wait()
            a = adjv_ref[m]
            sym = jnp.where(rows >= cols, a, a.T)
            sa = jnp.abs(sym)
            c = jnp.where(sa > _THRESH, sa, jnp.float32(0.0))
            deg = jnp.sum(c, axis=1) + 1.0
            dinv = jnp.where(deg > 0.0, jax.lax.rsqrt(deg), jnp.float32(0.0))
            ah_ref[m] = (dinv[:, None] * (c + eye) * dinv[None, :]).astype(bf)

    xc = x_ref[:].reshape(_PB * _S, _D).astype(bf)
    ys = []
    for m in range(_M):
        h = jnp.dot(xc, ws_ref[m], preferred_element_type=jnp.float32).astype(bf)
        hcat = jnp.concatenate([h[i * _S:(i + 1) * _S] for i in range(_PB)], axis=1)
        t = jnp.dot(ah_ref[m], hcat, preferred_element_type=jnp.float32)
        bm = (b0_ref if m == 0 else b1_ref)[0][None, :]
        y = jnp.maximum(t + bm, 0.0).astype(bf)
        ys.append(y)
    for p in range(_PB):
        sl = slice(p * _D, (p + 1) * _D)
        o = jnp.dot(ys[0][:, sl], wob_ref[0:_D], preferred_element_type=jnp.float32)
        o += jnp.dot(ys[1][:, sl], wob_ref[_D:2 * _D], preferred_element_type=jnp.float32)
        o += bo_ref[0][None, :]
        out_ref[p] = jnp.maximum(o, 0.0)


def kernel(x, adj, W_0_0, b_0_0, W_0_1, b_0_1, W_1_0, b_1_0, W_1_1, b_1_1,
           W_out, b_out):
    B = x.shape[0]
    b0 = jnp.tile((b_0_0 + b_0_1).reshape(1, _D), (1, _PB))
    b1 = jnp.tile((b_1_0 + b_1_1).reshape(1, _D), (1, _PB))
    bo = b_out.reshape(1, _D)
    const3 = lambda *_: (0, 0, 0)
    const2 = lambda *_: (0, 0)
    return pl.pallas_call(
        _gconv_body,
        grid=(B // _PB,),
        in_specs=[
            pl.BlockSpec((_PB, _S, _D), lambda j: (j, 0, 0)),
            pl.BlockSpec(memory_space=pltpu.HBM),
            pl.BlockSpec((_D, _D), const2),
            pl.BlockSpec((_D, _D), const2),
            pl.BlockSpec((_D, _D), const2),
            pl.BlockSpec((_D, _D), const2),
            pl.BlockSpec((1, _PB * _D), const2),
            pl.BlockSpec((1, _PB * _D), const2),
            pl.BlockSpec((_M * _D, _D), const2),
            pl.BlockSpec((1, _D), const2),
        ],
        out_specs=pl.BlockSpec((_PB, _S, _D), lambda j: (j, 0, 0)),
        out_shape=jax.ShapeDtypeStruct((B, _S, _D), jnp.float32),
        scratch_shapes=[
            pltpu.VMEM((_M, _S, _S), jnp.bfloat16),
            pltpu.VMEM((_M, _D, _D), jnp.bfloat16),
            pltpu.VMEM((_M * _D, _D), jnp.bfloat16),
            pltpu.VMEM((_M, _S, _S), jnp.float32),
            pltpu.SemaphoreType.DMA,
            pltpu.SemaphoreType.DMA,
        ],
    )(x, adj, W_0_0, W_0_1, W_1_0, W_1_1, b0, b1, W_out, bo)
